# Initial kernel scaffold; baseline (speedup 1.0000x reference)
import jax
import jax.numpy as jnp
from jax.experimental import pallas as pl


def kernel(patch, pos_table):
    raise NotImplementedError("write your pallas kernel here")


# speedup vs baseline: 1.0314x; 1.0314x over previous
import jax
import jax.numpy as jnp
from jax.experimental import pallas as pl


def _add_kernel(patch_ref, table_ref, out_ref):
    out_ref[...] = patch_ref[...] + table_ref[...]


def kernel(patch, pos_table):
    batch, num_patches, proj_dim = patch.shape
    block_b = 4
    grid = (batch // block_b,)
    return pl.pallas_call(
        _add_kernel,
        grid=grid,
        in_specs=[
            pl.BlockSpec((block_b, num_patches, proj_dim), lambda b: (b, 0, 0)),
            pl.BlockSpec((num_patches, proj_dim), lambda b: (0, 0)),
        ],
        out_specs=pl.BlockSpec((block_b, num_patches, proj_dim), lambda b: (b, 0, 0)),
        out_shape=jax.ShapeDtypeStruct(patch.shape, patch.dtype),
    )(patch, pos_table)


# speedup vs baseline: 1.0494x; 1.0175x over previous
import jax
import jax.numpy as jnp
from jax.experimental import pallas as pl


def _add_kernel(patch_ref, table_ref, out_ref):
    out_ref[...] = patch_ref[...] + table_ref[...]


def kernel(patch, pos_table):
    batch, num_patches, proj_dim = patch.shape
    block_b = 8
    grid = (batch // block_b,)
    return pl.pallas_call(
        _add_kernel,
        grid=grid,
        in_specs=[
            pl.BlockSpec((block_b, num_patches, proj_dim), lambda b: (b, 0, 0)),
            pl.BlockSpec((num_patches, proj_dim), lambda b: (0, 0)),
        ],
        out_specs=pl.BlockSpec((block_b, num_patches, proj_dim), lambda b: (b, 0, 0)),
        out_shape=jax.ShapeDtypeStruct(patch.shape, patch.dtype),
    )(patch, pos_table)
